# Initial kernel scaffold; baseline (speedup 1.0000x reference)
import jax
import jax.numpy as jnp
from jax.experimental import pallas as pl


def kernel(x, embedding_table):
    raise NotImplementedError("write your pallas kernel here")


# speedup vs baseline: 1.6486x; 1.6486x over previous
import functools

import jax
import jax.numpy as jnp
from jax import lax
from jax.experimental import pallas as pl
from jax.experimental.pallas import tpu as pltpu
from jax.experimental.pallas import tpu_sc as plsc


@functools.lru_cache(maxsize=None)
def _broadcast_rows(B, L, D, dtype_name):
    dtype = jnp.dtype(dtype_name)
    info = plsc.get_sparse_core_info()
    NC, NS = info.num_cores, info.num_subcores
    NW = NC * NS
    assert L % NW == 0
    rows_per_w = L // NW
    R = min(rows_per_w, 64)
    n_chunks = rows_per_w // R
    mesh = plsc.VectorSubcoreMesh(core_axis_name="c", subcore_axis_name="s")

    @functools.partial(
        pl.kernel,
        mesh=mesh,
        out_type=jax.ShapeDtypeStruct((B, L, D), dtype),
        scratch_types=[
            pltpu.VMEM((R, D), dtype),
        ],
    )
    def k(table_hbm, out_hbm, buf):
        wid = lax.axis_index("s") * NC + lax.axis_index("c")
        base = wid * rows_per_w

        def body(i, carry):
            off = base + i * R
            pltpu.sync_copy(table_hbm.at[pl.ds(off, R)], buf)
            for b in range(B):
                pltpu.sync_copy(buf, out_hbm.at[b, pl.ds(off, R)])
            return carry

        lax.fori_loop(0, n_chunks, body, 0)

    return k


def kernel(x, embedding_table):
    B, L, D = x.shape
    return _broadcast_rows(B, L, D, str(embedding_table.dtype))(embedding_table)
